# Initial kernel scaffold; baseline (speedup 1.0000x reference)
import functools

import jax
import jax.numpy as jnp
from jax import lax
from jax.experimental import pallas as pl
from jax.experimental.pallas import tpu as pltpu
from jax.experimental.pallas import tpu_sc as plsc

_NTOKENS = 100000
_NINP = 128
_P_E = 0.1
_P_I = 0.65

_B = 1024
_L = 200

_NC = 2
_NS = 16
_NW = _NC * _NS

_CHUNK = 128
_CPB = _B // _CHUNK
_NCH = _L * _CPB
_CPW = _NCH // _NW


def _sc_body(xt_hbm, maske_hbm, weight_hbm, mi_hbm, out_hbm,
             idx_v, mval_v, rows_v, out_v, mi_v,
             sem_mval, sem_row0, sem_row1, sem_out0, sem_out1):
    wid = lax.axis_index("s") * _NC + lax.axis_index("c")
    sem_row = (sem_row0, sem_row1)
    sem_out = (sem_out0, sem_out1)

    pltpu.sync_copy(xt_hbm.at[pl.ds(wid * _CPW, _CPW)], idx_v)

    for j in range(_CPW):
        pltpu.async_copy(maske_hbm.at[idx_v.at[j]], mval_v.at[j], sem_mval)
    pltpu.async_copy(weight_hbm.at[idx_v.at[0]], rows_v.at[0], sem_row[0])
    pltpu.async_copy(weight_hbm.at[idx_v.at[1]], rows_v.at[1], sem_row[1])

    pltpu.sync_copy(mi_hbm, mi_v)

    for j in range(_CPW):
        pltpu.make_async_copy(maske_hbm.at[idx_v.at[0]], mval_v.at[0],
                              sem_mval).wait()

    def iter_body(j2, carry):
        for b in range(2):
            c = 2 * j2 + b
            fr = wid * _CPW + c
            l = fr >> 3
            b0 = (fr & 7) * _CHUNK

            pltpu.make_async_copy(weight_hbm.at[idx_v.at[0]], rows_v.at[b],
                                  sem_row[b]).wait()

            @pl.when(j2 > 0)
            def _():
                pltpu.make_async_copy(out_v.at[b],
                                      out_hbm.at[pl.ds(0, _CHUNK), 0],
                                      sem_out[b]).wait()

            mrow = [mi_v[l, pl.ds(16 * d, 16)] for d in range(8)]

            def row_body(r, acc):
                s = mval_v[c, r]
                for d in range(8):
                    sl = pl.ds(16 * d, 16)
                    out_v[b, r, sl] = rows_v[b, r, sl] * s * mrow[d]
                return acc

            lax.fori_loop(0, _CHUNK, row_body, 0)

            @pl.when(j2 < _CPW // 2 - 1)
            def _():
                pltpu.async_copy(weight_hbm.at[idx_v.at[c + 2]], rows_v.at[b],
                                 sem_row[b])

            pltpu.async_copy(out_v.at[b], out_hbm.at[pl.ds(b0, _CHUNK), l],
                             sem_out[b])
        return carry

    lax.fori_loop(0, _CPW // 2, iter_body, 0)

    for b in range(2):
        pltpu.make_async_copy(out_v.at[b], out_hbm.at[pl.ds(0, _CHUNK), 0],
                              sem_out[b]).wait()


_launch = functools.partial(
    pl.kernel,
    mesh=plsc.VectorSubcoreMesh(core_axis_name="c", subcore_axis_name="s"),
    out_type=jax.ShapeDtypeStruct((_B, _L, _NINP), jnp.float32),
    scratch_types=[
        pltpu.VMEM((_CPW, _CHUNK), jnp.int32),
        pltpu.VMEM((_CPW, _CHUNK), jnp.float32),
        pltpu.VMEM((2, _CHUNK, _NINP), jnp.float32),
        pltpu.VMEM((2, _CHUNK, _NINP), jnp.float32),
        pltpu.VMEM((_L, _NINP), jnp.float32),
        pltpu.SemaphoreType.DMA,
        pltpu.SemaphoreType.DMA,
        pltpu.SemaphoreType.DMA,
        pltpu.SemaphoreType.DMA,
        pltpu.SemaphoreType.DMA,
    ],
)(_sc_body)


@jax.jit
def kernel(X, weight):
    mask_e = jax.random.bernoulli(
        jax.random.key(1), 1.0 - _P_E, (_NTOKENS, 1)).astype(weight.dtype)
    mask_e = (mask_e / (1.0 - _P_E))[:, 0]
    mask_i = jax.random.bernoulli(
        jax.random.key(2), 1.0 - _P_I, (1, _L, _NINP)).astype(weight.dtype)
    mask_i = (mask_i / (1.0 - _P_I))[0]
    xt = X.T.reshape(_NCH, _CHUNK)
    return _launch(xt, mask_e, weight, mask_i)


# speedup vs baseline: 4.2889x; 4.2889x over previous
import functools

import jax
import jax.numpy as jnp
from jax import lax
from jax.experimental import pallas as pl
from jax.experimental.pallas import tpu as pltpu
from jax.experimental.pallas import tpu_sc as plsc

_NTOKENS = 100000
_NINP = 128
_P_E = 0.1
_P_I = 0.65

_B = 1024
_L = 200

_NC = 2
_NS = 16
_NW = _NC * _NS

_CHUNK = 128
_CPB = _B // _CHUNK
_NCH = _L * _CPB
_CPW = _NCH // _NW


def _sc_body(xt_hbm, maske_hbm, weight_hbm, mi_hbm, out_hbm,
             idx_v, mval_v, rows_v, out_v, mi_v,
             sem_mval, sem_row0, sem_row1, sem_out0, sem_out1):
    wid = lax.axis_index("s") * _NC + lax.axis_index("c")
    sem_row = (sem_row0, sem_row1)
    sem_out = (sem_out0, sem_out1)

    pltpu.sync_copy(xt_hbm.at[wid], idx_v)

    for j in range(_CPW):
        pltpu.async_copy(maske_hbm.at[idx_v.at[j]], mval_v.at[j], sem_mval)
    pltpu.async_copy(weight_hbm.at[idx_v.at[0]], rows_v.at[0], sem_row[0])
    pltpu.async_copy(weight_hbm.at[idx_v.at[1]], rows_v.at[1], sem_row[1])

    pltpu.sync_copy(mi_hbm, mi_v)

    for j in range(_CPW):
        pltpu.make_async_copy(maske_hbm.at[idx_v.at[0]], mval_v.at[0],
                              sem_mval).wait()

    def iter_body(j2, carry):
        for b in range(2):
            c = 2 * j2 + b
            fr = wid * _CPW + c
            l = fr >> 3
            b0 = (fr & 7) * _CHUNK

            pltpu.make_async_copy(weight_hbm.at[idx_v.at[0]], rows_v.at[b],
                                  sem_row[b]).wait()

            @pl.when(j2 > 0)
            def _():
                pltpu.make_async_copy(out_v.at[b],
                                      out_hbm.at[pl.ds(0, _CHUNK),
                                                 pl.ds(0, _NINP)],
                                      sem_out[b]).wait()

            mrow = [mi_v[l, pl.ds(16 * d, 16)] for d in range(8)]

            def grp_body(g, acc):
                sv = mval_v[c, pl.ds(16 * g, 16)]
                for r16 in range(16):
                    r = 16 * g + r16
                    s = sv[r16]
                    for d in range(8):
                        sl = pl.ds(16 * d, 16)
                        out_v[b, r, sl] = rows_v[b, r, sl] * s * mrow[d]
                return acc

            lax.fori_loop(0, _CHUNK // 16, grp_body, 0)

            @pl.when(j2 < _CPW // 2 - 1)
            def _():
                pltpu.async_copy(weight_hbm.at[idx_v.at[c + 2]], rows_v.at[b],
                                 sem_row[b])

            pltpu.async_copy(out_v.at[b],
                             out_hbm.at[pl.ds(b0, _CHUNK),
                                        pl.ds(l * _NINP, _NINP)],
                             sem_out[b])
        return carry

    lax.fori_loop(0, _CPW // 2, iter_body, 0)

    for b in range(2):
        pltpu.make_async_copy(out_v.at[b],
                              out_hbm.at[pl.ds(0, _CHUNK), pl.ds(0, _NINP)],
                              sem_out[b]).wait()


_launch = functools.partial(
    pl.kernel,
    mesh=plsc.VectorSubcoreMesh(core_axis_name="c", subcore_axis_name="s"),
    out_type=jax.ShapeDtypeStruct((_B, _L * _NINP), jnp.float32),
    scratch_types=[
        pltpu.VMEM((_CPW, _CHUNK), jnp.int32),
        pltpu.VMEM((_CPW, _CHUNK), jnp.float32),
        pltpu.VMEM((2, _CHUNK, _NINP), jnp.float32),
        pltpu.VMEM((2, _CHUNK, _NINP), jnp.float32),
        pltpu.VMEM((_L, _NINP), jnp.float32),
        pltpu.SemaphoreType.DMA,
        pltpu.SemaphoreType.DMA,
        pltpu.SemaphoreType.DMA,
        pltpu.SemaphoreType.DMA,
        pltpu.SemaphoreType.DMA,
    ],
)(_sc_body)


@jax.jit
def kernel(X, weight):
    mask_e = jax.random.bernoulli(
        jax.random.key(1), 1.0 - _P_E, (_NTOKENS, 1)).astype(weight.dtype)
    mask_e = (mask_e / (1.0 - _P_E))[:, 0]
    mask_i = jax.random.bernoulli(
        jax.random.key(2), 1.0 - _P_I, (1, _L, _NINP)).astype(weight.dtype)
    mask_i = (mask_i / (1.0 - _P_I))[0]
    xt = X.T.reshape(_NW, _CPW, _CHUNK)
    out2 = _launch(xt, mask_e, weight, mask_i)
    return out2.reshape(_B, _L, _NINP)


# speedup vs baseline: 7.2593x; 1.6926x over previous
import functools

import jax
import jax.numpy as jnp
from jax import lax
from jax.experimental import pallas as pl
from jax.experimental.pallas import tpu as pltpu
from jax.experimental.pallas import tpu_sc as plsc

_NTOKENS = 100000
_NINP = 128
_P_E = 0.1
_P_I = 0.65

_B = 1024
_L = 200

_NC = 2
_NS = 16
_NW = _NC * _NS

_CHUNK = 128
_CPB = _B // _CHUNK
_NCH = _L * _CPB
_CPW = _NCH // _NW


def _sc_body(xt_hbm, maske_hbm, weight_hbm, mi_hbm, out_hbm,
             idx_v, mval_v, rows_v, out_v, oidx_v, mi_v,
             sem_mval, sem_row0, sem_row1, sem_out0, sem_out1):
    wid = lax.axis_index("s") * _NC + lax.axis_index("c")
    sem_row = (sem_row0, sem_row1)
    sem_out = (sem_out0, sem_out1)

    pltpu.sync_copy(xt_hbm.at[wid], idx_v)

    for j in range(_CPW):
        pltpu.async_copy(maske_hbm.at[idx_v.at[j]], mval_v.at[j], sem_mval)
    pltpu.async_copy(weight_hbm.at[idx_v.at[0]], rows_v.at[0], sem_row[0])
    pltpu.async_copy(weight_hbm.at[idx_v.at[1]], rows_v.at[1], sem_row[1])

    pltpu.sync_copy(mi_hbm, mi_v)

    for j in range(_CPW):
        pltpu.make_async_copy(maske_hbm.at[idx_v.at[0]], mval_v.at[0],
                              sem_mval).wait()

    def iter_body(j2, carry):
        for b in range(2):
            c = 2 * j2 + b
            fr = wid * _CPW + c
            l = fr >> 3
            b0 = (fr & 7) * _CHUNK

            pltpu.make_async_copy(weight_hbm.at[idx_v.at[0]], rows_v.at[b],
                                  sem_row[b]).wait()

            @pl.when(j2 > 0)
            def _():
                pltpu.make_async_copy(out_v.at[b], out_hbm.at[oidx_v.at[b]],
                                      sem_out[b]).wait()

            mrow = [mi_v[l, pl.ds(16 * d, 16)] for d in range(8)]

            def grp_body(g, acc):
                sv = mval_v[c, pl.ds(16 * g, 16)]
                for r16 in range(16):
                    r = 16 * g + r16
                    s = sv[r16]
                    for d in range(8):
                        sl = pl.ds(16 * d, 16)
                        out_v[b, r, sl] = rows_v[b, r, sl] * s * mrow[d]
                return acc

            lax.fori_loop(0, _CHUNK // 16, grp_body, 0)

            @pl.when(j2 < _CPW // 2 - 1)
            def _():
                pltpu.async_copy(weight_hbm.at[idx_v.at[c + 2]], rows_v.at[b],
                                 sem_row[b])

            lane = lax.iota(jnp.int32, 16)
            for k in range(8):
                oidx_v[b, pl.ds(16 * k, 16)] = (b0 + 16 * k + lane) * _L + l
            pltpu.async_copy(out_v.at[b], out_hbm.at[oidx_v.at[b]],
                             sem_out[b])
        return carry

    lax.fori_loop(0, _CPW // 2, iter_body, 0)

    for b in range(2):
        pltpu.make_async_copy(out_v.at[b], out_hbm.at[oidx_v.at[b]],
                              sem_out[b]).wait()


_launch = functools.partial(
    pl.kernel,
    mesh=plsc.VectorSubcoreMesh(core_axis_name="c", subcore_axis_name="s"),
    out_type=jax.ShapeDtypeStruct((_B * _L, _NINP), jnp.float32),
    scratch_types=[
        pltpu.VMEM((_CPW, _CHUNK), jnp.int32),
        pltpu.VMEM((_CPW, _CHUNK), jnp.float32),
        pltpu.VMEM((2, _CHUNK, _NINP), jnp.float32),
        pltpu.VMEM((2, _CHUNK, _NINP), jnp.float32),
        pltpu.VMEM((2, _CHUNK), jnp.int32),
        pltpu.VMEM((_L, _NINP), jnp.float32),
        pltpu.SemaphoreType.DMA,
        pltpu.SemaphoreType.DMA,
        pltpu.SemaphoreType.DMA,
        pltpu.SemaphoreType.DMA,
        pltpu.SemaphoreType.DMA,
    ],
)(_sc_body)


@jax.jit
def kernel(X, weight):
    mask_e = jax.random.bernoulli(
        jax.random.key(1), 1.0 - _P_E, (_NTOKENS, 1)).astype(weight.dtype)
    mask_e = (mask_e / (1.0 - _P_E))[:, 0]
    mask_i = jax.random.bernoulli(
        jax.random.key(2), 1.0 - _P_I, (1, _L, _NINP)).astype(weight.dtype)
    mask_i = (mask_i / (1.0 - _P_I))[0]
    xt = X.T.reshape(_NW, _CPW, _CHUNK)
    out2 = _launch(xt, mask_e, weight, mask_i)
    return out2.reshape(_B, _L, _NINP)
